# baseline (speedup 1.0000x reference)
import functools
import math

import jax
import jax.numpy as jnp
from jax import lax
from jax.experimental import pallas as pl
from jax.experimental.pallas import tpu as pltpu
from jax.experimental.pallas import tpu_sc as plsc

N = 10000
E = 320000
D = 128
H = 8
HD = 16
ED = 16

NCORES = 2
NSUB = 16
NW = NCORES * NSUB
C = 32
N1 = 10240
EPW = 10240
EPAD = NW * EPW
EPADX = EPAD + 2048
CHUNKS = EPW // C
SUP = 256
SUPT = SUP + C
RPT = N1 // NSUB
NS1 = N1 // 8


def _proj_body(xb, wq, wk, wv, bq, bk, bv, qo, ko, vo):
    xx = xb[...]
    qo[...] = jnp.dot(xx, wq[...], preferred_element_type=jnp.float32) + bq[...]
    ko[...] = jnp.dot(xx, wk[...], preferred_element_type=jnp.float32) + bk[...]
    vo[...] = jnp.dot(xx, wv[...], preferred_element_type=jnp.float32) + bv[...]


def _project_qkv(x, Wq, Wk, Wv, bq, bk, bv):
    blk = 256
    grid = (N1 // blk,)
    full = pl.BlockSpec((D, D), lambda i: (0, 0))
    vec = pl.BlockSpec((1, D), lambda i: (0, 0))
    row = pl.BlockSpec((blk, D), lambda i: (i, 0))
    return pl.pallas_call(
        _proj_body,
        grid=grid,
        in_specs=[row, full, full, full, vec, vec, vec],
        out_specs=[row, row, row],
        out_shape=[jax.ShapeDtypeStruct((N1, D), jnp.float32)] * 3,
    )(x, Wq, Wk, Wv, bq.reshape(1, D), bk.reshape(1, D), bv.reshape(1, D))


def _ebias_body(ea, we, be, out):
    out[...] = jnp.dot(ea[...], we[...], preferred_element_type=jnp.float32) + be[...]


def _edge_bias(ea, We16, be16):
    blk = 2048
    grid = (E // blk + 1,)
    return pl.pallas_call(
        _ebias_body,
        grid=grid,
        in_specs=[
            pl.BlockSpec((blk, ED), lambda i: (i, 0)),
            pl.BlockSpec((ED, 16), lambda i: (0, 0)),
            pl.BlockSpec((1, 16), lambda i: (0, 0)),
        ],
        out_specs=pl.BlockSpec((blk, 16), lambda i: (i, 0)),
        out_shape=jax.ShapeDtypeStruct((EPADX, 16), jnp.float32),
    )(ea, We16, be16.reshape(1, 16))


def _sc_body(q_hbm, k_hbm, v_hbm, dst_hbm, src_hbm, eb_hbm,
             outp_hbm, sp_hbm,
             dst0, src0, dst1, src1, rid_v, rid_s, rid_c, dstS, srcS, ebS,
             qi0, kj0, vj0, qi1, kj1, vj1, exs, exd,
             acc_o, acc_s, sem0, sem1):
    cid = lax.axis_index("c")
    sid = lax.axis_index("s")
    wid = cid * NSUB + sid
    lanes = lax.iota(jnp.int32, 16)
    zeros16 = jnp.zeros((16,), jnp.float32)

    def _zrow(r, carry):
        for c8 in range(D // 16):
            vj0[r, pl.ds(c8 * 16, 16)] = zeros16
        return carry

    lax.fori_loop(0, C, _zrow, 0)
    for r in range(16):
        for c8 in range(D // 16):
            exd[r, pl.ds(c8 * 16, 16)] = zeros16

    tbase = sid * RPT

    def _set_rid(base):
        for j in range(C // 16):
            rid_v[pl.ds(j * 16, 16)] = jnp.full((16,), base + j * 16,
                                                jnp.int32) + lanes

    def _set_rid_s(base):
        rid_s[...] = jnp.full((16,), base, jnp.int32) + lanes

    for i in range(RPT // C):
        _set_rid(tbase + i * C)
        pltpu.sync_copy(vj0, acc_o.at[rid_v])
    sbase = sid * (NS1 // NSUB)
    for i in range(NS1 // NSUB // 16):
        _set_rid_s(sbase + i * 16)
        pltpu.sync_copy(exd, acc_s.at[rid_s])
    plsc.subcore_barrier()

    ebase = wid * EPW

    def _copy_idx(koff, dstb, srcb):
        for j in range(C // 16):
            dstb[pl.ds(j * 16, 16)] = dstS[pl.ds(koff + j * 16, 16)]
            srcb[pl.ds(j * 16, 16)] = srcS[pl.ds(koff + j * 16, 16)]

    def _issue(dstb, srcb, qib, kjb, vjb, sem):
        pltpu.async_copy(q_hbm.at[dstb], qib, sem)
        pltpu.async_copy(k_hbm.at[srcb], kjb, sem)
        pltpu.async_copy(v_hbm.at[srcb], vjb, sem)

    def _drain(dstb, srcb, qib, kjb, vjb, sem):
        pltpu.make_async_copy(q_hbm.at[dstb], qib, sem).wait()
        pltpu.make_async_copy(k_hbm.at[srcb], kjb, sem).wait()
        pltpu.make_async_copy(v_hbm.at[srcb], vjb, sem).wait()

    def _compute(koff, dstb, qib, kjb, vjb):
        for r in range(C):
            for c8 in range(D // 16):
                exs[r, pl.ds(c8 * 16, 16)] = zeros16

        def _group(g, gcarry):
            dstg = dstb[pl.ds(g * 16, 16)]
            for ee in range(16):
                e = g * 16 + ee
                ebrow = ebS[pl.ds((koff + e) * 16, 16)]
                attnv = jnp.zeros((16,), jnp.float32)
                for h in range(H):
                    a = qib[e, pl.ds(h * HD, HD)]
                    b = kjb[e, pl.ds(h * HD, HD)]
                    dot = jnp.sum(a * b)
                    attnv = jnp.where(lanes == h, jnp.full((16,), dot), attnv)
                exv = jnp.exp(attnv * 0.25 + ebrow)
                for h in range(H):
                    ex_h = jnp.full((16,), exv[h])
                    vjb[e, pl.ds(h * HD, HD)] = vjb[e, pl.ds(h * HD, HD)] * ex_h
                dste = dstg[ee]
                exs[e, pl.ds((dste & 7) * 16, HD)] = exv
            return gcarry

        lax.fori_loop(0, C // 16, _group, 0)
        for j in range(C // 16):
            rid_c[pl.ds(j * 16, 16)] = lax.shift_right_logical(
                dstb[pl.ds(j * 16, 16)], 3)
        pltpu.sync_copy(exs, acc_s.at[rid_c], add=True)
        pltpu.sync_copy(vjb, acc_o.at[dstb], add=True)

    pltpu.sync_copy(dst_hbm.at[pl.ds(ebase, SUPT)], dstS)
    pltpu.sync_copy(src_hbm.at[pl.ds(ebase, SUPT)], srcS)
    _copy_idx(0, dst0, src0)
    _issue(dst0, src0, qi0, kj0, vj0, sem0)

    def _super(s, carry):
        sb = ebase + s * SUP
        pltpu.sync_copy(dst_hbm.at[pl.ds(sb, SUPT)], dstS)
        pltpu.sync_copy(src_hbm.at[pl.ds(sb, SUPT)], srcS)
        pltpu.sync_copy(eb_hbm.at[pl.ds(sb * 16, SUP * 16)], ebS)

        def _pair(q, qcarry):
            koffa = q * (2 * C)
            koffb = koffa + C
            _copy_idx(koffb, dst1, src1)
            _issue(dst1, src1, qi1, kj1, vj1, sem1)
            _drain(dst0, src0, qi0, kj0, vj0, sem0)
            _compute(koffa, dst0, qi0, kj0, vj0)
            _copy_idx(koffb + C, dst0, src0)
            _issue(dst0, src0, qi0, kj0, vj0, sem0)
            _drain(dst1, src1, qi1, kj1, vj1, sem1)
            _compute(koffb, dst1, qi1, kj1, vj1)
            return qcarry

        lax.fori_loop(0, SUP // (2 * C), _pair, 0)
        return carry

    lax.fori_loop(0, EPW // SUP, _super, 0)
    _drain(dst0, src0, qi0, kj0, vj0, sem0)
    plsc.subcore_barrier()

    for i in range(RPT // C):
        _set_rid(tbase + i * C)
        pltpu.sync_copy(acc_o.at[rid_v], vj0)
        pltpu.sync_copy(vj0, outp_hbm.at[cid, pl.ds(tbase + i * C, C)])
    for i in range(NS1 // NSUB // 16):
        _set_rid_s(sbase + i * 16)
        pltpu.sync_copy(acc_s.at[rid_s], exd)
        pltpu.sync_copy(exd, sp_hbm.at[cid, pl.ds(sbase + i * 16, 16)])


_sc_kernel = functools.partial(
    pl.kernel,
    mesh=plsc.VectorSubcoreMesh(core_axis_name="c", subcore_axis_name="s"),
    out_type=[
        jax.ShapeDtypeStruct((NCORES, N1, D), jnp.float32),
        jax.ShapeDtypeStruct((NCORES, NS1, D), jnp.float32),
    ],
    scratch_types=[
        pltpu.VMEM((C,), jnp.int32),
        pltpu.VMEM((C,), jnp.int32),
        pltpu.VMEM((C,), jnp.int32),
        pltpu.VMEM((C,), jnp.int32),
        pltpu.VMEM((C,), jnp.int32),
        pltpu.VMEM((16,), jnp.int32),
        pltpu.VMEM((C,), jnp.int32),
        pltpu.VMEM((SUPT,), jnp.int32),
        pltpu.VMEM((SUPT,), jnp.int32),
        pltpu.VMEM((SUP * 16,), jnp.float32),
        pltpu.VMEM((C, D), jnp.float32),
        pltpu.VMEM((C, D), jnp.float32),
        pltpu.VMEM((C, D), jnp.float32),
        pltpu.VMEM((C, D), jnp.float32),
        pltpu.VMEM((C, D), jnp.float32),
        pltpu.VMEM((C, D), jnp.float32),
        pltpu.VMEM((C, D), jnp.float32),
        pltpu.VMEM((16, D), jnp.float32),
        pltpu.VMEM_SHARED((N1, D), jnp.float32),
        pltpu.VMEM_SHARED((NS1, D), jnp.float32),
        pltpu.SemaphoreType.DMA,
        pltpu.SemaphoreType.DMA,
    ],
    compiler_params=pltpu.CompilerParams(needs_layout_passes=False),
)(_sc_body)


def _final_body(p0, p1, s0, s1, xb, expand, wo, bo, gm, bt, out):
    den = jnp.dot(s0[...] + s1[...] + 1e-8, expand[...],
                  preferred_element_type=jnp.float32)
    o = (p0[...] + p1[...]) / den
    y = jnp.dot(o, wo[...], preferred_element_type=jnp.float32) + bo[...] + xb[...]
    mu = jnp.mean(y, axis=-1, keepdims=True)
    var = jnp.mean((y - mu) ** 2, axis=-1, keepdims=True)
    out[...] = (y - mu) / jnp.sqrt(var + 1e-5) * gm[...] + bt[...]


def _finalize(p0, p1, s0, s1, x, expand, Wo, bo, gamma, beta):
    blk = 200
    grid = (N // blk,)
    row = pl.BlockSpec((blk, D), lambda i: (i, 0))
    srow = pl.BlockSpec((blk, H), lambda i: (i, 0))
    vec = pl.BlockSpec((1, D), lambda i: (0, 0))
    return pl.pallas_call(
        _final_body,
        grid=grid,
        in_specs=[row, row, srow, srow, row,
                  pl.BlockSpec((H, D), lambda i: (0, 0)),
                  pl.BlockSpec((D, D), lambda i: (0, 0)),
                  vec, vec, vec],
        out_specs=row,
        out_shape=jax.ShapeDtypeStruct((N, D), jnp.float32),
    )(p0, p1, s0, s1, x, expand, Wo,
      bo.reshape(1, D), gamma.reshape(1, D), beta.reshape(1, D))


def kernel(x, edge_index, edge_attr, Wq, bq, Wk, bk, Wv, bv, We, be, Wo, bo,
           gamma, beta):
    src = edge_index[0]
    dst = edge_index[1]
    pad_e = EPADX - E
    dst_p = jnp.concatenate([dst, jnp.full((pad_e,), N, jnp.int32)])
    src_p = jnp.concatenate([src, jnp.zeros((pad_e,), jnp.int32)])
    We16 = jnp.zeros((ED, 16), jnp.float32).at[:, :H].set(We)
    be16 = jnp.full((16,), -1e4, jnp.float32).at[:H].set(be)

    q, k, v = _project_qkv(x, Wq, Wk, Wv, bq, bk, bv)
    eb = _edge_bias(edge_attr, We16, be16).reshape(EPADX * 16)
    outp, sp = _sc_kernel(q, k, v, dst_p, src_p, eb)

    s8 = sp.reshape(NCORES, NS1, 8, 16)[..., :H].reshape(NCORES, N1, H)
    expand = jnp.repeat(jnp.eye(H, dtype=jnp.float32), HD, axis=1)
    return _finalize(outp[0], outp[1], s8[0], s8[1], x,
                     expand, Wo, bo, gamma, beta)


# speedup vs baseline: 1.0442x; 1.0442x over previous
import functools
import math

import jax
import jax.numpy as jnp
from jax import lax
from jax.experimental import pallas as pl
from jax.experimental.pallas import tpu as pltpu
from jax.experimental.pallas import tpu_sc as plsc

N = 10000
E = 320000
D = 128
H = 8
HD = 16
ED = 16

NCORES = 2
NSUB = 16
NW = NCORES * NSUB
C = 32
N1 = 10240
EPW = 10240
EPAD = NW * EPW
EPADX = EPAD + 2048
CHUNKS = EPW // C
SUP = 256
SUPT = SUP + C
RPT = N1 // NSUB
NS1 = N1 // 8


def _proj_body(xb, wq, wk, wv, bq, bk, bv, qo, ko, vo):
    xx = xb[...]
    qo[...] = jnp.dot(xx, wq[...], preferred_element_type=jnp.float32) + bq[...]
    ko[...] = jnp.dot(xx, wk[...], preferred_element_type=jnp.float32) + bk[...]
    vo[...] = jnp.dot(xx, wv[...], preferred_element_type=jnp.float32) + bv[...]


def _project_qkv(x, Wq, Wk, Wv, bq, bk, bv):
    blk = 256
    grid = (N1 // blk,)
    full = pl.BlockSpec((D, D), lambda i: (0, 0))
    vec = pl.BlockSpec((1, D), lambda i: (0, 0))
    row = pl.BlockSpec((blk, D), lambda i: (i, 0))
    return pl.pallas_call(
        _proj_body,
        grid=grid,
        in_specs=[row, full, full, full, vec, vec, vec],
        out_specs=[row, row, row],
        out_shape=[jax.ShapeDtypeStruct((N1, D), jnp.float32)] * 3,
    )(x, Wq, Wk, Wv, bq.reshape(1, D), bk.reshape(1, D), bv.reshape(1, D))


def _ebias_body(ea, we, be, out):
    out[...] = jnp.dot(ea[...], we[...], preferred_element_type=jnp.float32) + be[...]


def _edge_bias(ea, We16, be16):
    blk = 2048
    grid = (E // blk + 1,)
    return pl.pallas_call(
        _ebias_body,
        grid=grid,
        in_specs=[
            pl.BlockSpec((blk, ED), lambda i: (i, 0)),
            pl.BlockSpec((ED, 16), lambda i: (0, 0)),
            pl.BlockSpec((1, 16), lambda i: (0, 0)),
        ],
        out_specs=pl.BlockSpec((blk, 16), lambda i: (i, 0)),
        out_shape=jax.ShapeDtypeStruct((EPADX, 16), jnp.float32),
    )(ea, We16, be16.reshape(1, 16))


def _sc_body(q_hbm, k_hbm, v_hbm, dst_hbm, src_hbm, eb_hbm,
             outp_hbm, sp_hbm,
             dst0, src0, dst1, src1, rid_v, rid_s, dstS, srcS, ebS,
             qi0, kj0, vj0, qi1, kj1, vj1, exs,
             acc_o, acc_s, sem0, sem1):
    cid = lax.axis_index("c")
    sid = lax.axis_index("s")
    wid = cid * NSUB + sid
    lanes = lax.iota(jnp.int32, 16)
    zeros16 = jnp.zeros((16,), jnp.float32)

    def _zrow(r, carry):
        for c8 in range(D // 16):
            vj0[r, pl.ds(c8 * 16, 16)] = zeros16
        return carry

    lax.fori_loop(0, C, _zrow, 0)
    for r in range(16):
        for c8 in range(D // 16):
            exs[r, pl.ds(c8 * 16, 16)] = zeros16

    tbase = sid * RPT

    def _set_rid(base):
        for j in range(C // 16):
            rid_v[pl.ds(j * 16, 16)] = jnp.full((16,), base + j * 16,
                                                jnp.int32) + lanes

    def _set_rid_s(base):
        rid_s[...] = jnp.full((16,), base, jnp.int32) + lanes

    for i in range(RPT // C):
        _set_rid(tbase + i * C)
        pltpu.sync_copy(vj0, acc_o.at[rid_v])
    sbase = sid * (NS1 // NSUB)
    for i in range(NS1 // NSUB // 16):
        _set_rid_s(sbase + i * 16)
        pltpu.sync_copy(exs, acc_s.at[rid_s])
    plsc.subcore_barrier()

    ebase = wid * EPW

    def _copy_idx(koff, dstb, srcb):
        for j in range(C // 16):
            dstb[pl.ds(j * 16, 16)] = dstS[pl.ds(koff + j * 16, 16)]
            srcb[pl.ds(j * 16, 16)] = srcS[pl.ds(koff + j * 16, 16)]

    def _issue(dstb, srcb, qib, kjb, vjb, sem):
        pltpu.async_copy(q_hbm.at[dstb], qib, sem)
        pltpu.async_copy(k_hbm.at[srcb], kjb, sem)
        pltpu.async_copy(v_hbm.at[srcb], vjb, sem)

    def _drain(dstb, srcb, qib, kjb, vjb, sem):
        pltpu.make_async_copy(q_hbm.at[dstb], qib, sem).wait()
        pltpu.make_async_copy(k_hbm.at[srcb], kjb, sem).wait()
        pltpu.make_async_copy(v_hbm.at[srcb], vjb, sem).wait()

    def _compute(koff, dstb, qib, kjb, vjb):
        def _group(g, gcarry):
            for r in range(16):
                for c8 in range(D // 16):
                    exs[r, pl.ds(c8 * 16, 16)] = zeros16

            dstg = dstb[pl.ds(g * 16, 16)]
            for ee in range(16):
                e = g * 16 + ee
                ebrow = ebS[pl.ds((koff + e) * 16, 16)]
                attnv = jnp.zeros((16,), jnp.float32)
                for h in range(H):
                    a = qib[e, pl.ds(h * HD, HD)]
                    b = kjb[e, pl.ds(h * HD, HD)]
                    dot = jnp.sum(a * b)
                    attnv = jnp.where(lanes == h, jnp.full((16,), dot), attnv)
                exv = jnp.exp(attnv * 0.25 + ebrow)
                for h in range(H):
                    ex_h = jnp.full((16,), exv[h])
                    vjb[e, pl.ds(h * HD, HD)] = vjb[e, pl.ds(h * HD, HD)] * ex_h
                dste = dstg[ee]
                exs[ee, pl.ds((dste & 7) * 16, HD)] = exv

            rid_s[...] = lax.shift_right_logical(dstg, 3)
            pltpu.sync_copy(exs, acc_s.at[rid_s], add=True)
            return gcarry

        lax.fori_loop(0, C // 16, _group, 0)
        pltpu.sync_copy(vjb, acc_o.at[dstb], add=True)

    pltpu.sync_copy(dst_hbm.at[pl.ds(ebase, SUPT)], dstS)
    pltpu.sync_copy(src_hbm.at[pl.ds(ebase, SUPT)], srcS)
    _copy_idx(0, dst0, src0)
    _issue(dst0, src0, qi0, kj0, vj0, sem0)

    def _super(s, carry):
        sb = ebase + s * SUP
        pltpu.sync_copy(dst_hbm.at[pl.ds(sb, SUPT)], dstS)
        pltpu.sync_copy(src_hbm.at[pl.ds(sb, SUPT)], srcS)
        pltpu.sync_copy(eb_hbm.at[pl.ds(sb * 16, SUP * 16)], ebS)

        def _pair(q, qcarry):
            koffa = q * (2 * C)
            koffb = koffa + C
            _copy_idx(koffb, dst1, src1)
            _issue(dst1, src1, qi1, kj1, vj1, sem1)
            _drain(dst0, src0, qi0, kj0, vj0, sem0)
            _compute(koffa, dst0, qi0, kj0, vj0)
            _copy_idx(koffb + C, dst0, src0)
            _issue(dst0, src0, qi0, kj0, vj0, sem0)
            _drain(dst1, src1, qi1, kj1, vj1, sem1)
            _compute(koffb, dst1, qi1, kj1, vj1)
            return qcarry

        lax.fori_loop(0, SUP // (2 * C), _pair, 0)
        return carry

    lax.fori_loop(0, EPW // SUP, _super, 0)
    _drain(dst0, src0, qi0, kj0, vj0, sem0)
    plsc.subcore_barrier()

    for i in range(RPT // C):
        _set_rid(tbase + i * C)
        pltpu.sync_copy(acc_o.at[rid_v], vj0)
        pltpu.sync_copy(vj0, outp_hbm.at[cid, pl.ds(tbase + i * C, C)])
    for i in range(NS1 // NSUB // 16):
        _set_rid_s(sbase + i * 16)
        pltpu.sync_copy(acc_s.at[rid_s], exs)
        pltpu.sync_copy(exs, sp_hbm.at[cid, pl.ds(sbase + i * 16, 16)])


_sc_kernel = functools.partial(
    pl.kernel,
    mesh=plsc.VectorSubcoreMesh(core_axis_name="c", subcore_axis_name="s"),
    out_type=[
        jax.ShapeDtypeStruct((NCORES, N1, D), jnp.float32),
        jax.ShapeDtypeStruct((NCORES, NS1, D), jnp.float32),
    ],
    scratch_types=[
        pltpu.VMEM((C,), jnp.int32),
        pltpu.VMEM((C,), jnp.int32),
        pltpu.VMEM((C,), jnp.int32),
        pltpu.VMEM((C,), jnp.int32),
        pltpu.VMEM((C,), jnp.int32),
        pltpu.VMEM((16,), jnp.int32),
        pltpu.VMEM((SUPT,), jnp.int32),
        pltpu.VMEM((SUPT,), jnp.int32),
        pltpu.VMEM((SUP * 16,), jnp.float32),
        pltpu.VMEM((C, D), jnp.float32),
        pltpu.VMEM((C, D), jnp.float32),
        pltpu.VMEM((C, D), jnp.float32),
        pltpu.VMEM((C, D), jnp.float32),
        pltpu.VMEM((C, D), jnp.float32),
        pltpu.VMEM((C, D), jnp.float32),
        pltpu.VMEM((16, D), jnp.float32),
        pltpu.VMEM_SHARED((N1, D), jnp.float32),
        pltpu.VMEM_SHARED((NS1, D), jnp.float32),
        pltpu.SemaphoreType.DMA,
        pltpu.SemaphoreType.DMA,
    ],
    compiler_params=pltpu.CompilerParams(needs_layout_passes=False),
)(_sc_body)


def _final_body(p0, p1, s0, s1, xb, expand, wo, bo, gm, bt, out):
    den = jnp.dot(s0[...] + s1[...] + 1e-8, expand[...],
                  preferred_element_type=jnp.float32)
    o = (p0[...] + p1[...]) / den
    y = jnp.dot(o, wo[...], preferred_element_type=jnp.float32) + bo[...] + xb[...]
    mu = jnp.mean(y, axis=-1, keepdims=True)
    var = jnp.mean((y - mu) ** 2, axis=-1, keepdims=True)
    out[...] = (y - mu) / jnp.sqrt(var + 1e-5) * gm[...] + bt[...]


def _finalize(p0, p1, s0, s1, x, expand, Wo, bo, gamma, beta):
    blk = 200
    grid = (N // blk,)
    row = pl.BlockSpec((blk, D), lambda i: (i, 0))
    srow = pl.BlockSpec((blk, H), lambda i: (i, 0))
    vec = pl.BlockSpec((1, D), lambda i: (0, 0))
    return pl.pallas_call(
        _final_body,
        grid=grid,
        in_specs=[row, row, srow, srow, row,
                  pl.BlockSpec((H, D), lambda i: (0, 0)),
                  pl.BlockSpec((D, D), lambda i: (0, 0)),
                  vec, vec, vec],
        out_specs=row,
        out_shape=jax.ShapeDtypeStruct((N, D), jnp.float32),
    )(p0, p1, s0, s1, x, expand, Wo,
      bo.reshape(1, D), gamma.reshape(1, D), beta.reshape(1, D))


def kernel(x, edge_index, edge_attr, Wq, bq, Wk, bk, Wv, bv, We, be, Wo, bo,
           gamma, beta):
    src = edge_index[0]
    dst = edge_index[1]
    pad_e = EPADX - E
    dst_p = jnp.concatenate([dst, jnp.full((pad_e,), N, jnp.int32)])
    src_p = jnp.concatenate([src, jnp.zeros((pad_e,), jnp.int32)])
    We16 = jnp.zeros((ED, 16), jnp.float32).at[:, :H].set(We)
    be16 = jnp.full((16,), -1e4, jnp.float32).at[:H].set(be)

    q, k, v = _project_qkv(x, Wq, Wk, Wv, bq, bk, bv)
    eb = _edge_bias(edge_attr, We16, be16).reshape(EPADX * 16)
    outp, sp = _sc_kernel(q, k, v, dst_p, src_p, eb)

    s8 = sp.reshape(NCORES, NS1, 8, 16)[..., :H].reshape(NCORES, N1, H)
    expand = jnp.repeat(jnp.eye(H, dtype=jnp.float32), HD, axis=1)
    return _finalize(outp[0], outp[1], s8[0], s8[1], x,
                     expand, Wo, bo, gamma, beta)
